# baseline (speedup 1.0000x reference)
import functools

import jax
import jax.numpy as jnp
from jax import lax
from jax.experimental import pallas as pl
from jax.experimental.pallas import tpu as pltpu
from jax.experimental.pallas import tpu_sc as plsc

N = 10000
NP = 10240
E = 320000
D = 128
H1 = 32
H2 = 64
G = 64
CH = 128
NCHT = E // CH
PCH = NCHT // 32
XTRA = NCHT - 32 * PCH
RPT = NP // 16
NBUF = 6
BLK = 1000
NBLK = N // BLK

_F32 = jnp.float32
_BF16 = jnp.bfloat16
_HIGH = lax.Precision.HIGHEST


def _dot_bf16(a, b):
    return jnp.dot(a.astype(_BF16), b.astype(_BF16),
                   preferred_element_type=_F32)


def _mesh():
    return plsc.VectorSubcoreMesh(core_axis_name="c", subcore_axis_name="s",
                                  num_cores=2, num_subcores=16)


_SC_PARAMS = pltpu.CompilerParams(use_tc_tiling_on_sc=False)


@functools.cache
def _make_deg_sc():
    @functools.partial(
        pl.kernel,
        mesh=_mesh(),
        out_type=jax.ShapeDtypeStruct((2, NP), _F32),
        compiler_params=_SC_PARAMS,
        scratch_types=[
            pltpu.VMEM((PCH + 1, CH), jnp.int32),
            pltpu.VMEM((RPT,), _F32),
            pltpu.VMEM((CH,), _F32),
            pltpu.VMEM_SHARED((NP,), _F32),
            pltpu.SemaphoreType.DMA,
        ],
    )
    def _deg_sc(edge_hbm, out_hbm, dstb, zbuf, ones, acc, sem):
        c = lax.axis_index("c")
        s = lax.axis_index("s")
        wid = s * 2 + c
        idx_cp = pltpu.async_copy(edge_hbm.at[1, pl.ds(wid * PCH, PCH), :],
                                  dstb.at[pl.ds(0, PCH)], sem)
        z16 = jnp.zeros((16,), _F32)
        o16 = jnp.full((16,), 1.0, _F32)

        @pl.loop(0, RPT, step=16)
        def _(i):
            zbuf[pl.ds(i, 16)] = z16

        @pl.loop(0, CH, step=16)
        def _(i):
            ones[pl.ds(i, 16)] = o16

        pltpu.sync_copy(zbuf, acc.at[pl.ds(s * RPT, RPT)])
        idx_cp.wait()

        @pl.when(wid < XTRA)
        def _():
            pltpu.sync_copy(edge_hbm.at[1, pl.ds(32 * PCH + wid, 1), :],
                            dstb.at[pl.ds(PCH, 1)])

        plsc.subcore_barrier()

        @pl.loop(0, PCH)
        def _(j):
            pltpu.sync_copy(ones, acc.at[dstb.at[j]], add=True)

        @pl.when(wid < XTRA)
        def _():
            pltpu.sync_copy(ones, acc.at[dstb.at[PCH]], add=True)

        plsc.subcore_barrier()
        pltpu.sync_copy(acc.at[pl.ds(s * RPT, RPT)], zbuf)
        pltpu.sync_copy(zbuf, out_hbm.at[c, pl.ds(s * RPT, RPT)])

    return _deg_sc


@functools.cache
def _make_msg_sc(F):
    @functools.partial(
        pl.kernel,
        mesh=_mesh(),
        out_type=jax.ShapeDtypeStruct((2, NP, F), _F32),
        compiler_params=_SC_PARAMS,
        scratch_types=(
            [pltpu.VMEM((PCH + 1, CH), jnp.int32),
             pltpu.VMEM((PCH + 1, CH), jnp.int32)]
            + [pltpu.VMEM((CH, F), _F32) for _ in range(NBUF)]
            + [pltpu.SemaphoreType.DMA for _ in range(2 * NBUF)]
            + [pltpu.VMEM_SHARED((NP, F), _F32)]
        ),
    )
    def _msg(edge_hbm, tab_hbm, out_hbm, srcb, dstb, *rest):
        gbs = rest[:NBUF]
        gsems = rest[NBUF:2 * NBUF]
        ssems = rest[2 * NBUF:3 * NBUF]
        acc = rest[3 * NBUF]
        c = lax.axis_index("c")
        s = lax.axis_index("s")
        wid = s * 2 + c
        cp_s = pltpu.async_copy(edge_hbm.at[0, pl.ds(wid * PCH, PCH), :],
                                srcb.at[pl.ds(0, PCH)], gsems[0])
        cp_d = pltpu.async_copy(edge_hbm.at[1, pl.ds(wid * PCH, PCH), :],
                                dstb.at[pl.ds(0, PCH)], gsems[1])
        z16 = jnp.zeros((16,), _F32)

        @pl.loop(0, CH)
        def _(i):
            for jj in range(0, F, 16):
                gbs[0][i, pl.ds(jj, 16)] = z16

        @pl.loop(0, RPT, step=CH)
        def _(r):
            pltpu.sync_copy(gbs[0], acc.at[pl.ds(s * RPT + r, CH), :])

        cp_s.wait()
        cp_d.wait()

        @pl.when(wid < XTRA)
        def _():
            pltpu.sync_copy(edge_hbm.at[0, pl.ds(32 * PCH + wid, 1), :],
                            srcb.at[pl.ds(PCH, 1)])
            pltpu.sync_copy(edge_hbm.at[1, pl.ds(32 * PCH + wid, 1), :],
                            dstb.at[pl.ds(PCH, 1)])

        plsc.subcore_barrier()

        @pl.loop(0, PCH, step=NBUF)
        def _(j):
            hs = [pltpu.async_copy(tab_hbm.at[srcb.at[j + k]], gbs[k],
                                   gsems[k]) for k in range(NBUF)]
            ss = []
            for k in range(NBUF):
                hs[k].wait()
                ss.append(pltpu.async_copy(gbs[k], acc.at[dstb.at[j + k]],
                                           ssems[k], add=True))
            for k in range(NBUF):
                ss[k].wait()

        @pl.when(wid < XTRA)
        def _():
            pltpu.sync_copy(tab_hbm.at[srcb.at[PCH]], gbs[0])
            pltpu.sync_copy(gbs[0], acc.at[dstb.at[PCH]], add=True)

        plsc.subcore_barrier()

        @pl.loop(0, RPT, step=CH)
        def _(r):
            pltpu.sync_copy(acc.at[pl.ds(s * RPT + r, CH), :], gbs[0])
            pltpu.sync_copy(gbs[0], out_hbm.at[c, pl.ds(s * RPT + r, CH), :])

    return _msg


def _tc1_body(x_ref, w_ref, d0_ref, d1_ref, out_ref):
    deg = d0_ref[0] + d1_ref[0] + 1.0
    dinv = lax.rsqrt(jnp.maximum(deg, 1e-12))
    h = _dot_bf16(x_ref[...], w_ref[...])
    out_ref[...] = h * dinv


def _tc1(x, W1, degp3):
    return pl.pallas_call(
        _tc1_body,
        grid=(NBLK,),
        in_specs=[
            pl.BlockSpec((BLK, D), lambda i: (i, 0)),
            pl.BlockSpec((D, H1), lambda i: (0, 0)),
            pl.BlockSpec((1, BLK, 1), lambda i: (0, i, 0)),
            pl.BlockSpec((1, BLK, 1), lambda i: (1, i, 0)),
        ],
        out_specs=pl.BlockSpec((BLK, H1), lambda i: (i, 0)),
        out_shape=jax.ShapeDtypeStruct((N, H1), _F32),
    )(x, W1, degp3, degp3)


def _norm_relu(a0_ref, a1_ref, hp_ref, d0_ref, d1_ref, g_ref, be_ref, b_ref):
    deg = d0_ref[0] + d1_ref[0] + 1.0
    dinv = lax.rsqrt(jnp.maximum(deg, 1e-12))
    sfeat = (a0_ref[0] + a1_ref[0] + hp_ref[...]) * dinv + b_ref[...]
    mu = jnp.mean(sfeat, axis=1, keepdims=True)
    var = jnp.mean((sfeat - mu) ** 2, axis=1, keepdims=True)
    y = (sfeat - mu) * lax.rsqrt(var + 1e-5) * g_ref[...] + be_ref[...]
    return jnp.maximum(y, 0.0), dinv


def _tc2_body(a0_ref, a1_ref, hp_ref, d0_ref, d1_ref, g_ref, be_ref, b_ref,
              out_ref):
    y, dinv = _norm_relu(a0_ref, a1_ref, hp_ref, d0_ref, d1_ref,
                         g_ref, be_ref, b_ref)
    y16 = y.astype(_BF16).astype(_F32)
    out_ref[...] = y16 * dinv


def _tc2(accp1, hp1, degp3, g1, be1, b1):
    return pl.pallas_call(
        _tc2_body,
        grid=(NBLK,),
        in_specs=[
            pl.BlockSpec((1, BLK, H1), lambda i: (0, i, 0)),
            pl.BlockSpec((1, BLK, H1), lambda i: (1, i, 0)),
            pl.BlockSpec((BLK, H1), lambda i: (i, 0)),
            pl.BlockSpec((1, BLK, 1), lambda i: (0, i, 0)),
            pl.BlockSpec((1, BLK, 1), lambda i: (1, i, 0)),
            pl.BlockSpec((1, H1), lambda i: (0, 0)),
            pl.BlockSpec((1, H1), lambda i: (0, 0)),
            pl.BlockSpec((1, H1), lambda i: (0, 0)),
        ],
        out_specs=pl.BlockSpec((BLK, H1), lambda i: (i, 0)),
        out_shape=jax.ShapeDtypeStruct((N, H1), _F32),
    )(accp1, accp1, hp1, degp3, degp3, g1, be1, b1)


def _tc3_body(a0_ref, a1_ref, hp_ref, d0_ref, d1_ref, w2_ref, g_ref, be_ref,
              b_ref, bt_ref, wc1_ref, bc1_ref, wc2t_ref, bc2_ref, out_ref,
              sums_ref, cnts_ref):
    i = pl.program_id(0)
    deg = d0_ref[0] + d1_ref[0] + 1.0
    dinv = lax.rsqrt(jnp.maximum(deg, 1e-12))
    agg = (a0_ref[0] + a1_ref[0] + hp_ref[...]) * dinv
    w2b = w2_ref[...].astype(_BF16).astype(_F32)
    sfeat = jnp.dot(agg, w2b, preferred_element_type=_F32,
                    precision=_HIGH) + b_ref[...]
    mu = jnp.mean(sfeat, axis=1, keepdims=True)
    var = jnp.mean((sfeat - mu) ** 2, axis=1, keepdims=True)
    y = (sfeat - mu) * lax.rsqrt(var + 1e-5) * g_ref[...] + be_ref[...]
    y = jnp.maximum(y, 0.0)
    bb = bt_ref[0]
    gid = lax.broadcasted_iota(jnp.int32, (G, BLK), 0)
    oh = (gid == bb).astype(_F32)
    psum = jnp.dot(oh, y, preferred_element_type=_F32, precision=_HIGH)
    pcnt = jnp.sum(oh, axis=1, keepdims=True)

    @pl.when(i == 0)
    def _():
        sums_ref[...] = psum
        cnts_ref[...] = pcnt

    @pl.when(i > 0)
    def _():
        sums_ref[...] += psum
        cnts_ref[...] += pcnt

    @pl.when(i == NBLK - 1)
    def _():
        pooled = sums_ref[...] / jnp.maximum(cnts_ref[...], 1.0)
        z = _dot_bf16(pooled, wc1_ref[...]) + bc1_ref[...]
        z = jnp.maximum(z, 0.0)
        zb = z.astype(_BF16).astype(_F32)
        wb = wc2t_ref[...].astype(_BF16).astype(_F32)
        out_ref[...] = (jnp.sum(zb * wb, axis=1, keepdims=True)
                        + bc2_ref[...])


def _tc3(accp2, hp2, degp3, W2, g2, be2, b2, batch3, Wc1, bc1, Wc2t, bc2):
    return pl.pallas_call(
        _tc3_body,
        grid=(NBLK,),
        in_specs=[
            pl.BlockSpec((1, BLK, H1), lambda i: (0, i, 0)),
            pl.BlockSpec((1, BLK, H1), lambda i: (1, i, 0)),
            pl.BlockSpec((BLK, H1), lambda i: (i, 0)),
            pl.BlockSpec((1, BLK, 1), lambda i: (0, i, 0)),
            pl.BlockSpec((1, BLK, 1), lambda i: (1, i, 0)),
            pl.BlockSpec((H1, H2), lambda i: (0, 0)),
            pl.BlockSpec((1, H2), lambda i: (0, 0)),
            pl.BlockSpec((1, H2), lambda i: (0, 0)),
            pl.BlockSpec((1, H2), lambda i: (0, 0)),
            pl.BlockSpec((1, 1, BLK), lambda i: (i, 0, 0)),
            pl.BlockSpec((H2, 32), lambda i: (0, 0)),
            pl.BlockSpec((1, 32), lambda i: (0, 0)),
            pl.BlockSpec((1, 32), lambda i: (0, 0)),
            pl.BlockSpec((1, 1), lambda i: (0, 0)),
        ],
        out_specs=pl.BlockSpec((G, 1), lambda i: (0, 0)),
        out_shape=jax.ShapeDtypeStruct((G, 1), _F32),
        scratch_shapes=[
            pltpu.VMEM((G, H2), _F32),
            pltpu.VMEM((G, 1), _F32),
        ],
    )(accp2, accp2, hp2, degp3, degp3, W2, g2, be2, b2, batch3, Wc1, bc1,
      Wc2t, bc2)


def kernel(x, edge_index, batch, W1, b1, g1, be1, W2, b2, g2, be2,
           Wc1, bc1, Wc2, bc2):
    edge3 = edge_index.reshape(2, NCHT, CH)
    batch3 = batch.reshape(NBLK, 1, BLK)

    degp3 = _make_deg_sc()(edge3).reshape(2, NP, 1)
    hp1 = _tc1(x, W1, degp3)
    accp1 = _make_msg_sc(H1)(edge3, hp1)
    hp2 = _tc2(accp1, hp1, degp3, g1.reshape(1, H1), be1.reshape(1, H1),
               b1.reshape(1, H1))
    accp2 = _make_msg_sc(H1)(edge3, hp2)
    return _tc3(accp2, hp2, degp3, W2, g2.reshape(1, H2), be2.reshape(1, H2),
                b2.reshape(1, H2), batch3, Wc1, bc1.reshape(1, 32),
                Wc2.reshape(1, 32), bc2.reshape(1, 1))


# speedup vs baseline: 1.0795x; 1.0795x over previous
import functools

import jax
import jax.numpy as jnp
from jax import lax
from jax.experimental import pallas as pl
from jax.experimental.pallas import tpu as pltpu
from jax.experimental.pallas import tpu_sc as plsc

N = 10000
NP = 10240
E = 320000
D = 128
H1 = 32
H2 = 64
G = 64
CH = 128
NCHT = E // CH
PCH = NCHT // 32
XTRA = NCHT - 32 * PCH
RPT = NP // 16
NBUF = 6
BLK = 1024
NBLK = NP // BLK

_F32 = jnp.float32
_BF16 = jnp.bfloat16
_HIGH = lax.Precision.HIGHEST


def _dot_bf16(a, b):
    return jnp.dot(a.astype(_BF16), b.astype(_BF16),
                   preferred_element_type=_F32)


def _mesh():
    return plsc.VectorSubcoreMesh(core_axis_name="c", subcore_axis_name="s",
                                  num_cores=2, num_subcores=16)


_SC_PARAMS = pltpu.CompilerParams(use_tc_tiling_on_sc=False)


@functools.cache
def _make_deg_sc():
    @functools.partial(
        pl.kernel,
        mesh=_mesh(),
        out_type=jax.ShapeDtypeStruct((2, NP), _F32),
        compiler_params=_SC_PARAMS,
        scratch_types=[
            pltpu.VMEM((PCH + 1, CH), jnp.int32),
            pltpu.VMEM((RPT,), _F32),
            pltpu.VMEM((CH,), _F32),
            pltpu.VMEM_SHARED((NP,), _F32),
            pltpu.SemaphoreType.DMA,
        ],
    )
    def _deg_sc(edge_hbm, out_hbm, dstb, zbuf, ones, acc, sem):
        c = lax.axis_index("c")
        s = lax.axis_index("s")
        wid = s * 2 + c
        idx_cp = pltpu.async_copy(edge_hbm.at[1, pl.ds(wid * PCH, PCH), :],
                                  dstb.at[pl.ds(0, PCH)], sem)
        z16 = jnp.zeros((16,), _F32)
        o16 = jnp.full((16,), 1.0, _F32)

        @pl.loop(0, RPT, step=16)
        def _(i):
            zbuf[pl.ds(i, 16)] = z16

        @pl.loop(0, CH, step=16)
        def _(i):
            ones[pl.ds(i, 16)] = o16

        pltpu.sync_copy(zbuf, acc.at[pl.ds(s * RPT, RPT)])
        idx_cp.wait()

        @pl.when(wid < XTRA)
        def _():
            pltpu.sync_copy(edge_hbm.at[1, pl.ds(32 * PCH + wid, 1), :],
                            dstb.at[pl.ds(PCH, 1)])

        plsc.subcore_barrier()

        @pl.loop(0, PCH)
        def _(j):
            pltpu.sync_copy(ones, acc.at[dstb.at[j]], add=True)

        @pl.when(wid < XTRA)
        def _():
            pltpu.sync_copy(ones, acc.at[dstb.at[PCH]], add=True)

        plsc.subcore_barrier()
        pltpu.sync_copy(acc.at[pl.ds(s * RPT, RPT)], zbuf)
        pltpu.sync_copy(zbuf, out_hbm.at[c, pl.ds(s * RPT, RPT)])

    return _deg_sc


@functools.cache
def _make_msg_sc(F):
    @functools.partial(
        pl.kernel,
        mesh=_mesh(),
        out_type=jax.ShapeDtypeStruct((2, NP, F), _F32),
        compiler_params=_SC_PARAMS,
        scratch_types=(
            [pltpu.VMEM((PCH + 1, CH), jnp.int32),
             pltpu.VMEM((PCH + 1, CH), jnp.int32)]
            + [pltpu.VMEM((CH, F), _F32) for _ in range(NBUF)]
            + [pltpu.SemaphoreType.DMA for _ in range(2 * NBUF)]
            + [pltpu.VMEM_SHARED((NP, F), _F32)]
        ),
    )
    def _msg(edge_hbm, tab_hbm, out_hbm, srcb, dstb, *rest):
        gbs = rest[:NBUF]
        gsems = rest[NBUF:2 * NBUF]
        ssems = rest[2 * NBUF:3 * NBUF]
        acc = rest[3 * NBUF]
        c = lax.axis_index("c")
        s = lax.axis_index("s")
        wid = s * 2 + c
        cp_s = pltpu.async_copy(edge_hbm.at[0, pl.ds(wid * PCH, PCH), :],
                                srcb.at[pl.ds(0, PCH)], gsems[0])
        cp_d = pltpu.async_copy(edge_hbm.at[1, pl.ds(wid * PCH, PCH), :],
                                dstb.at[pl.ds(0, PCH)], gsems[1])
        z16 = jnp.zeros((16,), _F32)

        @pl.loop(0, CH)
        def _(i):
            for jj in range(0, F, 16):
                gbs[0][i, pl.ds(jj, 16)] = z16

        @pl.loop(0, RPT, step=CH)
        def _(r):
            pltpu.sync_copy(gbs[0], acc.at[pl.ds(s * RPT + r, CH), :])

        cp_s.wait()
        cp_d.wait()

        @pl.when(wid < XTRA)
        def _():
            pltpu.sync_copy(edge_hbm.at[0, pl.ds(32 * PCH + wid, 1), :],
                            srcb.at[pl.ds(PCH, 1)])
            pltpu.sync_copy(edge_hbm.at[1, pl.ds(32 * PCH + wid, 1), :],
                            dstb.at[pl.ds(PCH, 1)])

        plsc.subcore_barrier()

        @pl.loop(0, PCH, step=NBUF)
        def _(j):
            hs = [pltpu.async_copy(tab_hbm.at[srcb.at[j + k]], gbs[k],
                                   gsems[k]) for k in range(NBUF)]
            ss = []
            for k in range(NBUF):
                hs[k].wait()
                ss.append(pltpu.async_copy(gbs[k], acc.at[dstb.at[j + k]],
                                           ssems[k], add=True))
            for k in range(NBUF):
                ss[k].wait()

        @pl.when(wid < XTRA)
        def _():
            pltpu.sync_copy(tab_hbm.at[srcb.at[PCH]], gbs[0])
            pltpu.sync_copy(gbs[0], acc.at[dstb.at[PCH]], add=True)

        plsc.subcore_barrier()

        @pl.loop(0, RPT, step=CH)
        def _(r):
            pltpu.sync_copy(acc.at[pl.ds(s * RPT + r, CH), :], gbs[0])
            pltpu.sync_copy(gbs[0], out_hbm.at[c, pl.ds(s * RPT + r, CH), :])

    return _msg


def _dinv_col(d_ref):
    deg = d_ref[0:1, :] + d_ref[1:2, :] + 1.0
    dinv = lax.rsqrt(jnp.maximum(deg, 1e-12))
    return jnp.transpose(dinv, (1, 0))


def _tc1_body(x_ref, w_ref, d_ref, out_ref):
    dinv = _dinv_col(d_ref)
    h = _dot_bf16(x_ref[...], w_ref[...])
    out_ref[...] = h * dinv


def _tc1(x_pad, W1, degp):
    return pl.pallas_call(
        _tc1_body,
        grid=(NBLK,),
        in_specs=[
            pl.BlockSpec((BLK, D), lambda i: (i, 0)),
            pl.BlockSpec((D, H1), lambda i: (0, 0)),
            pl.BlockSpec((2, BLK), lambda i: (0, i)),
        ],
        out_specs=pl.BlockSpec((BLK, H1), lambda i: (i, 0)),
        out_shape=jax.ShapeDtypeStruct((NP, H1), _F32),
    )(x_pad, W1, degp)


def _norm_relu(a0_ref, a1_ref, hp_ref, d_ref, g_ref, be_ref, b_ref):
    dinv = _dinv_col(d_ref)
    sfeat = (a0_ref[0] + a1_ref[0] + hp_ref[...]) * dinv + b_ref[...]
    mu = jnp.mean(sfeat, axis=1, keepdims=True)
    var = jnp.mean((sfeat - mu) ** 2, axis=1, keepdims=True)
    y = (sfeat - mu) * lax.rsqrt(var + 1e-5) * g_ref[...] + be_ref[...]
    return jnp.maximum(y, 0.0), dinv


def _tc2_body(a0_ref, a1_ref, hp_ref, d_ref, g_ref, be_ref, b_ref,
              out_ref):
    y, dinv = _norm_relu(a0_ref, a1_ref, hp_ref, d_ref, g_ref, be_ref, b_ref)
    y16 = y.astype(_BF16).astype(_F32)
    out_ref[...] = y16 * dinv


def _tc2(accp1, hp1, degp, g1, be1, b1):
    return pl.pallas_call(
        _tc2_body,
        grid=(NBLK,),
        in_specs=[
            pl.BlockSpec((1, BLK, H1), lambda i: (0, i, 0)),
            pl.BlockSpec((1, BLK, H1), lambda i: (1, i, 0)),
            pl.BlockSpec((BLK, H1), lambda i: (i, 0)),
            pl.BlockSpec((2, BLK), lambda i: (0, i)),
            pl.BlockSpec((1, H1), lambda i: (0, 0)),
            pl.BlockSpec((1, H1), lambda i: (0, 0)),
            pl.BlockSpec((1, H1), lambda i: (0, 0)),
        ],
        out_specs=pl.BlockSpec((BLK, H1), lambda i: (i, 0)),
        out_shape=jax.ShapeDtypeStruct((NP, H1), _F32),
    )(accp1, accp1, hp1, degp, g1, be1, b1)


def _tc3_body(a0_ref, a1_ref, hp_ref, d_ref, w2_ref, g_ref, be_ref,
              b_ref, bt_ref, wc1_ref, bc1_ref, wc2t_ref, bc2_ref, out_ref,
              sums_ref, cnts_ref):
    i = pl.program_id(0)
    dinv = _dinv_col(d_ref)
    agg = (a0_ref[0] + a1_ref[0] + hp_ref[...]) * dinv
    w2b = w2_ref[...].astype(_BF16).astype(_F32)
    sfeat = jnp.dot(agg, w2b, preferred_element_type=_F32,
                    precision=_HIGH) + b_ref[...]
    mu = jnp.mean(sfeat, axis=1, keepdims=True)
    var = jnp.mean((sfeat - mu) ** 2, axis=1, keepdims=True)
    y = (sfeat - mu) * lax.rsqrt(var + 1e-5) * g_ref[...] + be_ref[...]
    y = jnp.maximum(y, 0.0)
    bb = bt_ref[0]
    gid = lax.broadcasted_iota(jnp.int32, (G, BLK), 0)
    oh = (gid == bb).astype(_F32)
    psum = jnp.dot(oh, y, preferred_element_type=_F32, precision=_HIGH)
    pcnt = jnp.sum(oh, axis=1, keepdims=True)

    @pl.when(i == 0)
    def _():
        sums_ref[...] = psum
        cnts_ref[...] = pcnt

    @pl.when(i > 0)
    def _():
        sums_ref[...] += psum
        cnts_ref[...] += pcnt

    @pl.when(i == NBLK - 1)
    def _():
        pooled = sums_ref[...] / jnp.maximum(cnts_ref[...], 1.0)
        z = _dot_bf16(pooled, wc1_ref[...]) + bc1_ref[...]
        z = jnp.maximum(z, 0.0)
        zb = z.astype(_BF16).astype(_F32)
        wb = wc2t_ref[...].astype(_BF16).astype(_F32)
        out_ref[...] = (jnp.sum(zb * wb, axis=1, keepdims=True)
                        + bc2_ref[...])


def _tc3(accp2, hp2, degp, W2, g2, be2, b2, batch3, Wc1, bc1, Wc2t, bc2):
    return pl.pallas_call(
        _tc3_body,
        grid=(NBLK,),
        in_specs=[
            pl.BlockSpec((1, BLK, H1), lambda i: (0, i, 0)),
            pl.BlockSpec((1, BLK, H1), lambda i: (1, i, 0)),
            pl.BlockSpec((BLK, H1), lambda i: (i, 0)),
            pl.BlockSpec((2, BLK), lambda i: (0, i)),
            pl.BlockSpec((H1, H2), lambda i: (0, 0)),
            pl.BlockSpec((1, H2), lambda i: (0, 0)),
            pl.BlockSpec((1, H2), lambda i: (0, 0)),
            pl.BlockSpec((1, H2), lambda i: (0, 0)),
            pl.BlockSpec((1, 1, BLK), lambda i: (i, 0, 0)),
            pl.BlockSpec((H2, 32), lambda i: (0, 0)),
            pl.BlockSpec((1, 32), lambda i: (0, 0)),
            pl.BlockSpec((1, 32), lambda i: (0, 0)),
            pl.BlockSpec((1, 1), lambda i: (0, 0)),
        ],
        out_specs=pl.BlockSpec((G, 1), lambda i: (0, 0)),
        out_shape=jax.ShapeDtypeStruct((G, 1), _F32),
        scratch_shapes=[
            pltpu.VMEM((G, H2), _F32),
            pltpu.VMEM((G, 1), _F32),
        ],
    )(accp2, accp2, hp2, degp, W2, g2, be2, b2, batch3, Wc1, bc1,
      Wc2t, bc2)


def kernel(x, edge_index, batch, W1, b1, g1, be1, W2, b2, g2, be2,
           Wc1, bc1, Wc2, bc2):
    edge3 = edge_index.reshape(2, NCHT, CH)
    x_pad = jnp.pad(x, ((0, NP - N), (0, 0)))
    batch3 = jnp.pad(batch, (0, NP - N),
                     constant_values=G).reshape(NBLK, 1, BLK)

    degp = _make_deg_sc()(edge3)
    hp1 = _tc1(x_pad, W1, degp)
    accp1 = _make_msg_sc(H1)(edge3, hp1)
    hp2 = _tc2(accp1, hp1, degp, g1.reshape(1, H1), be1.reshape(1, H1),
               b1.reshape(1, H1))
    accp2 = _make_msg_sc(H1)(edge3, hp2)
    return _tc3(accp2, hp2, degp, W2, g2.reshape(1, H2), be2.reshape(1, H2),
                b2.reshape(1, H2), batch3, Wc1, bc1.reshape(1, 32),
                Wc2.reshape(1, 32), bc2.reshape(1, 1))


# speedup vs baseline: 1.1199x; 1.0374x over previous
import functools

import jax
import jax.numpy as jnp
from jax import lax
from jax.experimental import pallas as pl
from jax.experimental.pallas import tpu as pltpu
from jax.experimental.pallas import tpu_sc as plsc

N = 10000
NP = 10240
E = 320000
D = 128
H1 = 32
H2 = 64
G = 64
CH = 128
NCHT = E // CH
PCH = NCHT // 32
XTRA = NCHT - 32 * PCH
RPT = NP // 16
NBUF = 6
BLK = 2048
NBLK = NP // BLK

_F32 = jnp.float32
_BF16 = jnp.bfloat16
_HIGH = lax.Precision.HIGHEST


def _dot_bf16(a, b):
    return jnp.dot(a.astype(_BF16), b.astype(_BF16),
                   preferred_element_type=_F32)


def _mesh():
    return plsc.VectorSubcoreMesh(core_axis_name="c", subcore_axis_name="s",
                                  num_cores=2, num_subcores=16)


_SC_PARAMS = pltpu.CompilerParams(use_tc_tiling_on_sc=False)


@functools.cache
def _make_deg_sc():
    @functools.partial(
        pl.kernel,
        mesh=_mesh(),
        out_type=jax.ShapeDtypeStruct((2, NP), _F32),
        compiler_params=_SC_PARAMS,
        scratch_types=[
            pltpu.VMEM((PCH + 1, CH), jnp.int32),
            pltpu.VMEM((RPT,), _F32),
            pltpu.VMEM((CH,), _F32),
            pltpu.VMEM_SHARED((NP,), _F32),
            pltpu.SemaphoreType.DMA,
        ],
    )
    def _deg_sc(edge_hbm, out_hbm, dstb, zbuf, ones, acc, sem):
        c = lax.axis_index("c")
        s = lax.axis_index("s")
        wid = s * 2 + c
        idx_cp = pltpu.async_copy(edge_hbm.at[1, pl.ds(wid * PCH, PCH), :],
                                  dstb.at[pl.ds(0, PCH)], sem)
        z16 = jnp.zeros((16,), _F32)
        o16 = jnp.full((16,), 1.0, _F32)

        @pl.loop(0, RPT, step=16)
        def _(i):
            zbuf[pl.ds(i, 16)] = z16

        @pl.loop(0, CH, step=16)
        def _(i):
            ones[pl.ds(i, 16)] = o16

        pltpu.sync_copy(zbuf, acc.at[pl.ds(s * RPT, RPT)])
        idx_cp.wait()

        @pl.when(wid < XTRA)
        def _():
            pltpu.sync_copy(edge_hbm.at[1, pl.ds(32 * PCH + wid, 1), :],
                            dstb.at[pl.ds(PCH, 1)])

        plsc.subcore_barrier()

        @pl.loop(0, PCH)
        def _(j):
            pltpu.sync_copy(ones, acc.at[dstb.at[j]], add=True)

        @pl.when(wid < XTRA)
        def _():
            pltpu.sync_copy(ones, acc.at[dstb.at[PCH]], add=True)

        plsc.subcore_barrier()
        pltpu.sync_copy(acc.at[pl.ds(s * RPT, RPT)], zbuf)
        pltpu.sync_copy(zbuf, out_hbm.at[c, pl.ds(s * RPT, RPT)])

    return _deg_sc


@functools.cache
def _make_msg_sc(F):
    @functools.partial(
        pl.kernel,
        mesh=_mesh(),
        out_type=jax.ShapeDtypeStruct((2, NP, F), _F32),
        compiler_params=_SC_PARAMS,
        scratch_types=(
            [pltpu.VMEM((PCH + 1, CH), jnp.int32),
             pltpu.VMEM((PCH + 1, CH), jnp.int32)]
            + [pltpu.VMEM((CH, F), _F32) for _ in range(NBUF)]
            + [pltpu.SemaphoreType.DMA for _ in range(2 * NBUF)]
            + [pltpu.VMEM_SHARED((NP, F), _F32)]
        ),
    )
    def _msg(edge_hbm, tab_hbm, out_hbm, srcb, dstb, *rest):
        gbs = rest[:NBUF]
        gsems = rest[NBUF:2 * NBUF]
        ssems = rest[2 * NBUF:3 * NBUF]
        acc = rest[3 * NBUF]
        c = lax.axis_index("c")
        s = lax.axis_index("s")
        wid = s * 2 + c
        cp_s = pltpu.async_copy(edge_hbm.at[0, pl.ds(wid * PCH, PCH), :],
                                srcb.at[pl.ds(0, PCH)], gsems[0])
        cp_d = pltpu.async_copy(edge_hbm.at[1, pl.ds(wid * PCH, PCH), :],
                                dstb.at[pl.ds(0, PCH)], gsems[1])
        z16 = jnp.zeros((16,), _F32)

        @pl.loop(0, CH)
        def _(i):
            for jj in range(0, F, 16):
                gbs[0][i, pl.ds(jj, 16)] = z16

        @pl.loop(0, RPT, step=CH)
        def _(r):
            pltpu.sync_copy(gbs[0], acc.at[pl.ds(s * RPT + r, CH), :])

        cp_s.wait()
        cp_d.wait()

        @pl.when(wid < XTRA)
        def _():
            pltpu.sync_copy(edge_hbm.at[0, pl.ds(32 * PCH + wid, 1), :],
                            srcb.at[pl.ds(PCH, 1)])
            pltpu.sync_copy(edge_hbm.at[1, pl.ds(32 * PCH + wid, 1), :],
                            dstb.at[pl.ds(PCH, 1)])

        plsc.subcore_barrier()

        @pl.loop(0, PCH, step=NBUF)
        def _(j):
            hs = [pltpu.async_copy(tab_hbm.at[srcb.at[j + k]], gbs[k],
                                   gsems[k]) for k in range(NBUF)]
            ss = []
            for k in range(NBUF):
                hs[k].wait()
                ss.append(pltpu.async_copy(gbs[k], acc.at[dstb.at[j + k]],
                                           ssems[k], add=True))
            for k in range(NBUF):
                ss[k].wait()

        @pl.when(wid < XTRA)
        def _():
            pltpu.sync_copy(tab_hbm.at[srcb.at[PCH]], gbs[0])
            pltpu.sync_copy(gbs[0], acc.at[dstb.at[PCH]], add=True)

        plsc.subcore_barrier()

        @pl.loop(0, RPT, step=CH)
        def _(r):
            pltpu.sync_copy(acc.at[pl.ds(s * RPT + r, CH), :], gbs[0])
            pltpu.sync_copy(gbs[0], out_hbm.at[c, pl.ds(s * RPT + r, CH), :])

    return _msg


def _dinv_col(d_ref):
    deg = d_ref[0:1, :] + d_ref[1:2, :] + 1.0
    dinv = lax.rsqrt(jnp.maximum(deg, 1e-12))
    return jnp.transpose(dinv, (1, 0))


def _tc1_body(x_ref, w_ref, d_ref, out_ref):
    dinv = _dinv_col(d_ref)
    h = _dot_bf16(x_ref[...], w_ref[...])
    out_ref[...] = h * dinv


def _tc1(x_pad, W1, degp):
    return pl.pallas_call(
        _tc1_body,
        grid=(NBLK,),
        in_specs=[
            pl.BlockSpec((BLK, D), lambda i: (i, 0)),
            pl.BlockSpec((D, H1), lambda i: (0, 0)),
            pl.BlockSpec((2, BLK), lambda i: (0, i)),
        ],
        out_specs=pl.BlockSpec((BLK, H1), lambda i: (i, 0)),
        out_shape=jax.ShapeDtypeStruct((NP, H1), _F32),
    )(x_pad, W1, degp)


def _norm_relu(a0_ref, a1_ref, hp_ref, d_ref, g_ref, be_ref, b_ref):
    dinv = _dinv_col(d_ref)
    sfeat = (a0_ref[0] + a1_ref[0] + hp_ref[...]) * dinv + b_ref[...]
    mu = jnp.mean(sfeat, axis=1, keepdims=True)
    var = jnp.mean((sfeat - mu) ** 2, axis=1, keepdims=True)
    y = (sfeat - mu) * lax.rsqrt(var + 1e-5) * g_ref[...] + be_ref[...]
    return jnp.maximum(y, 0.0), dinv


def _tc2_body(a0_ref, a1_ref, hp_ref, d_ref, g_ref, be_ref, b_ref,
              out_ref):
    y, dinv = _norm_relu(a0_ref, a1_ref, hp_ref, d_ref, g_ref, be_ref, b_ref)
    y16 = y.astype(_BF16).astype(_F32)
    out_ref[...] = y16 * dinv


def _tc2(accp1, hp1, degp, g1, be1, b1):
    return pl.pallas_call(
        _tc2_body,
        grid=(NBLK,),
        in_specs=[
            pl.BlockSpec((1, BLK, H1), lambda i: (0, i, 0)),
            pl.BlockSpec((1, BLK, H1), lambda i: (1, i, 0)),
            pl.BlockSpec((BLK, H1), lambda i: (i, 0)),
            pl.BlockSpec((2, BLK), lambda i: (0, i)),
            pl.BlockSpec((1, H1), lambda i: (0, 0)),
            pl.BlockSpec((1, H1), lambda i: (0, 0)),
            pl.BlockSpec((1, H1), lambda i: (0, 0)),
        ],
        out_specs=pl.BlockSpec((BLK, H1), lambda i: (i, 0)),
        out_shape=jax.ShapeDtypeStruct((NP, H1), _F32),
    )(accp1, accp1, hp1, degp, g1, be1, b1)


def _tc3_body(a0_ref, a1_ref, hp_ref, d_ref, w2_ref, g_ref, be_ref,
              b_ref, bt_ref, wc1_ref, bc1_ref, wc2t_ref, bc2_ref, out_ref,
              sums_ref, cnts_ref):
    i = pl.program_id(0)
    dinv = _dinv_col(d_ref)
    agg = (a0_ref[0] + a1_ref[0] + hp_ref[...]) * dinv
    w2b = w2_ref[...].astype(_BF16).astype(_F32)
    sfeat = jnp.dot(agg, w2b, preferred_element_type=_F32,
                    precision=_HIGH) + b_ref[...]
    mu = jnp.mean(sfeat, axis=1, keepdims=True)
    var = jnp.mean((sfeat - mu) ** 2, axis=1, keepdims=True)
    y = (sfeat - mu) * lax.rsqrt(var + 1e-5) * g_ref[...] + be_ref[...]
    y = jnp.maximum(y, 0.0)
    bb = bt_ref[0]
    gid = lax.broadcasted_iota(jnp.int32, (G, BLK), 0)
    oh = (gid == bb).astype(_F32)
    psum = jnp.dot(oh, y, preferred_element_type=_F32, precision=_HIGH)
    pcnt = jnp.sum(oh, axis=1, keepdims=True)

    @pl.when(i == 0)
    def _():
        sums_ref[...] = psum
        cnts_ref[...] = pcnt

    @pl.when(i > 0)
    def _():
        sums_ref[...] += psum
        cnts_ref[...] += pcnt

    @pl.when(i == NBLK - 1)
    def _():
        pooled = sums_ref[...] / jnp.maximum(cnts_ref[...], 1.0)
        z = _dot_bf16(pooled, wc1_ref[...]) + bc1_ref[...]
        z = jnp.maximum(z, 0.0)
        zb = z.astype(_BF16).astype(_F32)
        wb = wc2t_ref[...].astype(_BF16).astype(_F32)
        out_ref[...] = (jnp.sum(zb * wb, axis=1, keepdims=True)
                        + bc2_ref[...])


def _tc3(accp2, hp2, degp, W2, g2, be2, b2, batch3, Wc1, bc1, Wc2t, bc2):
    return pl.pallas_call(
        _tc3_body,
        grid=(NBLK,),
        in_specs=[
            pl.BlockSpec((1, BLK, H1), lambda i: (0, i, 0)),
            pl.BlockSpec((1, BLK, H1), lambda i: (1, i, 0)),
            pl.BlockSpec((BLK, H1), lambda i: (i, 0)),
            pl.BlockSpec((2, BLK), lambda i: (0, i)),
            pl.BlockSpec((H1, H2), lambda i: (0, 0)),
            pl.BlockSpec((1, H2), lambda i: (0, 0)),
            pl.BlockSpec((1, H2), lambda i: (0, 0)),
            pl.BlockSpec((1, H2), lambda i: (0, 0)),
            pl.BlockSpec((1, 1, BLK), lambda i: (i, 0, 0)),
            pl.BlockSpec((H2, 32), lambda i: (0, 0)),
            pl.BlockSpec((1, 32), lambda i: (0, 0)),
            pl.BlockSpec((1, 32), lambda i: (0, 0)),
            pl.BlockSpec((1, 1), lambda i: (0, 0)),
        ],
        out_specs=pl.BlockSpec((G, 1), lambda i: (0, 0)),
        out_shape=jax.ShapeDtypeStruct((G, 1), _F32),
        scratch_shapes=[
            pltpu.VMEM((G, H2), _F32),
            pltpu.VMEM((G, 1), _F32),
        ],
    )(accp2, accp2, hp2, degp, W2, g2, be2, b2, batch3, Wc1, bc1,
      Wc2t, bc2)


def kernel(x, edge_index, batch, W1, b1, g1, be1, W2, b2, g2, be2,
           Wc1, bc1, Wc2, bc2):
    edge3 = edge_index.reshape(2, NCHT, CH)
    x_pad = jnp.pad(x, ((0, NP - N), (0, 0)))
    batch3 = jnp.pad(batch, (0, NP - N),
                     constant_values=G).reshape(NBLK, 1, BLK)

    degp = _make_deg_sc()(edge3)
    hp1 = _tc1(x_pad, W1, degp)
    accp1 = _make_msg_sc(H1)(edge3, hp1)
    hp2 = _tc2(accp1, hp1, degp, g1.reshape(1, H1), be1.reshape(1, H1),
               b1.reshape(1, H1))
    accp2 = _make_msg_sc(H1)(edge3, hp2)
    return _tc3(accp2, hp2, degp, W2, g2.reshape(1, H2), be2.reshape(1, H2),
                b2.reshape(1, H2), batch3, Wc1, bc1.reshape(1, 32),
                Wc2.reshape(1, 32), bc2.reshape(1, 1))


# speedup vs baseline: 1.1262x; 1.0056x over previous
import functools

import jax
import jax.numpy as jnp
from jax import lax
from jax.experimental import pallas as pl
from jax.experimental.pallas import tpu as pltpu
from jax.experimental.pallas import tpu_sc as plsc

N = 10000
NP = 10240
E = 320000
D = 128
H1 = 32
H2 = 64
G = 64
CH = 128
NCHT = E // CH
PCH = NCHT // 32
XTRA = NCHT - 32 * PCH
RPT = NP // 16
NBUF = 6
BLK = 5120
NBLK = NP // BLK

_F32 = jnp.float32
_BF16 = jnp.bfloat16
_HIGH = lax.Precision.HIGHEST


def _dot_bf16(a, b):
    return jnp.dot(a.astype(_BF16), b.astype(_BF16),
                   preferred_element_type=_F32)


def _mesh():
    return plsc.VectorSubcoreMesh(core_axis_name="c", subcore_axis_name="s",
                                  num_cores=2, num_subcores=16)


_SC_PARAMS = pltpu.CompilerParams(use_tc_tiling_on_sc=False)


@functools.cache
def _make_deg_sc():
    @functools.partial(
        pl.kernel,
        mesh=_mesh(),
        out_type=jax.ShapeDtypeStruct((2, NP), _F32),
        compiler_params=_SC_PARAMS,
        scratch_types=[
            pltpu.VMEM((PCH + 1, CH), jnp.int32),
            pltpu.VMEM((RPT,), _F32),
            pltpu.VMEM((CH,), _F32),
            pltpu.VMEM_SHARED((NP,), _F32),
            pltpu.SemaphoreType.DMA,
        ],
    )
    def _deg_sc(edge_hbm, out_hbm, dstb, zbuf, ones, acc, sem):
        c = lax.axis_index("c")
        s = lax.axis_index("s")
        wid = s * 2 + c
        idx_cp = pltpu.async_copy(edge_hbm.at[1, pl.ds(wid * PCH, PCH), :],
                                  dstb.at[pl.ds(0, PCH)], sem)
        z16 = jnp.zeros((16,), _F32)
        o16 = jnp.full((16,), 1.0, _F32)

        @pl.loop(0, RPT, step=16)
        def _(i):
            zbuf[pl.ds(i, 16)] = z16

        @pl.loop(0, CH, step=16)
        def _(i):
            ones[pl.ds(i, 16)] = o16

        pltpu.sync_copy(zbuf, acc.at[pl.ds(s * RPT, RPT)])
        idx_cp.wait()

        @pl.when(wid < XTRA)
        def _():
            pltpu.sync_copy(edge_hbm.at[1, pl.ds(32 * PCH + wid, 1), :],
                            dstb.at[pl.ds(PCH, 1)])

        plsc.subcore_barrier()

        @pl.loop(0, PCH)
        def _(j):
            pltpu.sync_copy(ones, acc.at[dstb.at[j]], add=True)

        @pl.when(wid < XTRA)
        def _():
            pltpu.sync_copy(ones, acc.at[dstb.at[PCH]], add=True)

        plsc.subcore_barrier()
        pltpu.sync_copy(acc.at[pl.ds(s * RPT, RPT)], zbuf)
        pltpu.sync_copy(zbuf, out_hbm.at[c, pl.ds(s * RPT, RPT)])

    return _deg_sc


@functools.cache
def _make_msg_sc(F):
    @functools.partial(
        pl.kernel,
        mesh=_mesh(),
        out_type=jax.ShapeDtypeStruct((2, NP, F), _F32),
        compiler_params=_SC_PARAMS,
        scratch_types=(
            [pltpu.VMEM((PCH + 1, CH), jnp.int32),
             pltpu.VMEM((PCH + 1, CH), jnp.int32)]
            + [pltpu.VMEM((CH, F), _F32) for _ in range(NBUF)]
            + [pltpu.SemaphoreType.DMA for _ in range(2 * NBUF)]
            + [pltpu.VMEM_SHARED((NP, F), _F32)]
        ),
    )
    def _msg(edge_hbm, tab_hbm, out_hbm, srcb, dstb, *rest):
        gbs = rest[:NBUF]
        gsems = rest[NBUF:2 * NBUF]
        ssems = rest[2 * NBUF:3 * NBUF]
        acc = rest[3 * NBUF]
        c = lax.axis_index("c")
        s = lax.axis_index("s")
        wid = s * 2 + c
        cp_s = pltpu.async_copy(edge_hbm.at[0, pl.ds(wid * PCH, PCH), :],
                                srcb.at[pl.ds(0, PCH)], gsems[0])
        cp_d = pltpu.async_copy(edge_hbm.at[1, pl.ds(wid * PCH, PCH), :],
                                dstb.at[pl.ds(0, PCH)], gsems[1])
        z16 = jnp.zeros((16,), _F32)

        @pl.loop(0, CH)
        def _(i):
            for jj in range(0, F, 16):
                gbs[0][i, pl.ds(jj, 16)] = z16

        @pl.loop(0, RPT, step=CH)
        def _(r):
            pltpu.sync_copy(gbs[0], acc.at[pl.ds(s * RPT + r, CH), :])

        cp_s.wait()
        cp_d.wait()

        @pl.when(wid < XTRA)
        def _():
            pltpu.sync_copy(edge_hbm.at[0, pl.ds(32 * PCH + wid, 1), :],
                            srcb.at[pl.ds(PCH, 1)])
            pltpu.sync_copy(edge_hbm.at[1, pl.ds(32 * PCH + wid, 1), :],
                            dstb.at[pl.ds(PCH, 1)])

        plsc.subcore_barrier()

        @pl.loop(0, PCH, step=NBUF)
        def _(j):
            hs = [pltpu.async_copy(tab_hbm.at[srcb.at[j + k]], gbs[k],
                                   gsems[k]) for k in range(NBUF)]
            ss = []
            for k in range(NBUF):
                hs[k].wait()
                ss.append(pltpu.async_copy(gbs[k], acc.at[dstb.at[j + k]],
                                           ssems[k], add=True))
            for k in range(NBUF):
                ss[k].wait()

        @pl.when(wid < XTRA)
        def _():
            pltpu.sync_copy(tab_hbm.at[srcb.at[PCH]], gbs[0])
            pltpu.sync_copy(gbs[0], acc.at[dstb.at[PCH]], add=True)

        plsc.subcore_barrier()

        @pl.loop(0, RPT, step=CH)
        def _(r):
            pltpu.sync_copy(acc.at[pl.ds(s * RPT + r, CH), :], gbs[0])
            pltpu.sync_copy(gbs[0], out_hbm.at[c, pl.ds(s * RPT + r, CH), :])

    return _msg


def _dinv_col(d_ref):
    deg = d_ref[0:1, :] + d_ref[1:2, :] + 1.0
    dinv = lax.rsqrt(jnp.maximum(deg, 1e-12))
    return jnp.transpose(dinv, (1, 0))


def _tc1_body(x_ref, w_ref, d_ref, out_ref):
    dinv = _dinv_col(d_ref)
    h = _dot_bf16(x_ref[...], w_ref[...])
    out_ref[...] = h * dinv


def _tc1(x_pad, W1, degp):
    return pl.pallas_call(
        _tc1_body,
        grid=(NBLK,),
        in_specs=[
            pl.BlockSpec((BLK, D), lambda i: (i, 0)),
            pl.BlockSpec((D, H1), lambda i: (0, 0)),
            pl.BlockSpec((2, BLK), lambda i: (0, i)),
        ],
        out_specs=pl.BlockSpec((BLK, H1), lambda i: (i, 0)),
        out_shape=jax.ShapeDtypeStruct((NP, H1), _F32),
    )(x_pad, W1, degp)


def _norm_relu(a0_ref, a1_ref, hp_ref, d_ref, g_ref, be_ref, b_ref):
    dinv = _dinv_col(d_ref)
    sfeat = (a0_ref[0] + a1_ref[0] + hp_ref[...]) * dinv + b_ref[...]
    mu = jnp.mean(sfeat, axis=1, keepdims=True)
    var = jnp.mean((sfeat - mu) ** 2, axis=1, keepdims=True)
    y = (sfeat - mu) * lax.rsqrt(var + 1e-5) * g_ref[...] + be_ref[...]
    return jnp.maximum(y, 0.0), dinv


def _tc2_body(a0_ref, a1_ref, hp_ref, d_ref, g_ref, be_ref, b_ref,
              out_ref):
    y, dinv = _norm_relu(a0_ref, a1_ref, hp_ref, d_ref, g_ref, be_ref, b_ref)
    y16 = y.astype(_BF16).astype(_F32)
    out_ref[...] = y16 * dinv


def _tc2(accp1, hp1, degp, g1, be1, b1):
    return pl.pallas_call(
        _tc2_body,
        grid=(NBLK,),
        in_specs=[
            pl.BlockSpec((1, BLK, H1), lambda i: (0, i, 0)),
            pl.BlockSpec((1, BLK, H1), lambda i: (1, i, 0)),
            pl.BlockSpec((BLK, H1), lambda i: (i, 0)),
            pl.BlockSpec((2, BLK), lambda i: (0, i)),
            pl.BlockSpec((1, H1), lambda i: (0, 0)),
            pl.BlockSpec((1, H1), lambda i: (0, 0)),
            pl.BlockSpec((1, H1), lambda i: (0, 0)),
        ],
        out_specs=pl.BlockSpec((BLK, H1), lambda i: (i, 0)),
        out_shape=jax.ShapeDtypeStruct((NP, H1), _F32),
    )(accp1, accp1, hp1, degp, g1, be1, b1)


def _tc3_body(a0_ref, a1_ref, hp_ref, d_ref, w2_ref, g_ref, be_ref,
              b_ref, bt_ref, wc1_ref, bc1_ref, wc2t_ref, bc2_ref, out_ref,
              sums_ref, cnts_ref):
    i = pl.program_id(0)
    dinv = _dinv_col(d_ref)
    agg = (a0_ref[0] + a1_ref[0] + hp_ref[...]) * dinv
    w2b = w2_ref[...].astype(_BF16).astype(_F32)
    sfeat = jnp.dot(agg, w2b, preferred_element_type=_F32,
                    precision=_HIGH) + b_ref[...]
    mu = jnp.mean(sfeat, axis=1, keepdims=True)
    var = jnp.mean((sfeat - mu) ** 2, axis=1, keepdims=True)
    y = (sfeat - mu) * lax.rsqrt(var + 1e-5) * g_ref[...] + be_ref[...]
    y = jnp.maximum(y, 0.0)
    bb = bt_ref[0]
    gid = lax.broadcasted_iota(jnp.int32, (G, BLK), 0)
    oh = (gid == bb).astype(_F32)
    psum = jnp.dot(oh, y, preferred_element_type=_F32, precision=_HIGH)
    pcnt = jnp.sum(oh, axis=1, keepdims=True)

    @pl.when(i == 0)
    def _():
        sums_ref[...] = psum
        cnts_ref[...] = pcnt

    @pl.when(i > 0)
    def _():
        sums_ref[...] += psum
        cnts_ref[...] += pcnt

    @pl.when(i == NBLK - 1)
    def _():
        pooled = sums_ref[...] / jnp.maximum(cnts_ref[...], 1.0)
        z = _dot_bf16(pooled, wc1_ref[...]) + bc1_ref[...]
        z = jnp.maximum(z, 0.0)
        zb = z.astype(_BF16).astype(_F32)
        wb = wc2t_ref[...].astype(_BF16).astype(_F32)
        out_ref[...] = (jnp.sum(zb * wb, axis=1, keepdims=True)
                        + bc2_ref[...])


def _tc3(accp2, hp2, degp, W2, g2, be2, b2, batch3, Wc1, bc1, Wc2t, bc2):
    return pl.pallas_call(
        _tc3_body,
        grid=(NBLK,),
        in_specs=[
            pl.BlockSpec((1, BLK, H1), lambda i: (0, i, 0)),
            pl.BlockSpec((1, BLK, H1), lambda i: (1, i, 0)),
            pl.BlockSpec((BLK, H1), lambda i: (i, 0)),
            pl.BlockSpec((2, BLK), lambda i: (0, i)),
            pl.BlockSpec((H1, H2), lambda i: (0, 0)),
            pl.BlockSpec((1, H2), lambda i: (0, 0)),
            pl.BlockSpec((1, H2), lambda i: (0, 0)),
            pl.BlockSpec((1, H2), lambda i: (0, 0)),
            pl.BlockSpec((1, 1, BLK), lambda i: (i, 0, 0)),
            pl.BlockSpec((H2, 32), lambda i: (0, 0)),
            pl.BlockSpec((1, 32), lambda i: (0, 0)),
            pl.BlockSpec((1, 32), lambda i: (0, 0)),
            pl.BlockSpec((1, 1), lambda i: (0, 0)),
        ],
        out_specs=pl.BlockSpec((G, 1), lambda i: (0, 0)),
        out_shape=jax.ShapeDtypeStruct((G, 1), _F32),
        scratch_shapes=[
            pltpu.VMEM((G, H2), _F32),
            pltpu.VMEM((G, 1), _F32),
        ],
    )(accp2, accp2, hp2, degp, W2, g2, be2, b2, batch3, Wc1, bc1,
      Wc2t, bc2)


def kernel(x, edge_index, batch, W1, b1, g1, be1, W2, b2, g2, be2,
           Wc1, bc1, Wc2, bc2):
    edge3 = edge_index.reshape(2, NCHT, CH)
    x_pad = jnp.pad(x, ((0, NP - N), (0, 0)))
    batch3 = jnp.pad(batch, (0, NP - N),
                     constant_values=G).reshape(NBLK, 1, BLK)

    degp = _make_deg_sc()(edge3)
    hp1 = _tc1(x_pad, W1, degp)
    accp1 = _make_msg_sc(H1)(edge3, hp1)
    hp2 = _tc2(accp1, hp1, degp, g1.reshape(1, H1), be1.reshape(1, H1),
               b1.reshape(1, H1))
    accp2 = _make_msg_sc(H1)(edge3, hp2)
    return _tc3(accp2, hp2, degp, W2, g2.reshape(1, H2), be2.reshape(1, H2),
                b2.reshape(1, H2), batch3, Wc1, bc1.reshape(1, 32),
                Wc2.reshape(1, 32), bc2.reshape(1, 1))


# speedup vs baseline: 1.2022x; 1.0675x over previous
import functools

import jax
import jax.numpy as jnp
from jax import lax
from jax.experimental import pallas as pl
from jax.experimental.pallas import tpu as pltpu
from jax.experimental.pallas import tpu_sc as plsc

N = 10000
NP = 10240
E = 320000
D = 128
H1 = 32
H2 = 64
G = 64
CH = 128
NCHT = E // CH
PCH = NCHT // 32
XTRA = NCHT - 32 * PCH
RPT = NP // 16
NBUF = 13
BLK = 5120
NBLK = NP // BLK

_F32 = jnp.float32
_BF16 = jnp.bfloat16
_HIGH = lax.Precision.HIGHEST


def _dot_bf16(a, b):
    return jnp.dot(a.astype(_BF16), b.astype(_BF16),
                   preferred_element_type=_F32)


def _mesh():
    return plsc.VectorSubcoreMesh(core_axis_name="c", subcore_axis_name="s",
                                  num_cores=2, num_subcores=16)


_SC_PARAMS = pltpu.CompilerParams(use_tc_tiling_on_sc=False)


@functools.cache
def _make_deg_sc():
    @functools.partial(
        pl.kernel,
        mesh=_mesh(),
        out_type=jax.ShapeDtypeStruct((2, NP), _F32),
        compiler_params=_SC_PARAMS,
        scratch_types=[
            pltpu.VMEM((PCH + 1, CH), jnp.int32),
            pltpu.VMEM((RPT,), _F32),
            pltpu.VMEM((CH,), _F32),
            pltpu.VMEM_SHARED((NP,), _F32),
            pltpu.SemaphoreType.DMA,
        ],
    )
    def _deg_sc(edge_hbm, out_hbm, dstb, zbuf, ones, acc, sem):
        c = lax.axis_index("c")
        s = lax.axis_index("s")
        wid = s * 2 + c
        idx_cp = pltpu.async_copy(edge_hbm.at[1, pl.ds(wid * PCH, PCH), :],
                                  dstb.at[pl.ds(0, PCH)], sem)
        z16 = jnp.zeros((16,), _F32)
        o16 = jnp.full((16,), 1.0, _F32)

        @pl.loop(0, RPT, step=16)
        def _(i):
            zbuf[pl.ds(i, 16)] = z16

        @pl.loop(0, CH, step=16)
        def _(i):
            ones[pl.ds(i, 16)] = o16

        pltpu.sync_copy(zbuf, acc.at[pl.ds(s * RPT, RPT)])
        idx_cp.wait()

        @pl.when(wid < XTRA)
        def _():
            pltpu.sync_copy(edge_hbm.at[1, pl.ds(32 * PCH + wid, 1), :],
                            dstb.at[pl.ds(PCH, 1)])

        plsc.subcore_barrier()

        @pl.loop(0, PCH)
        def _(j):
            pltpu.sync_copy(ones, acc.at[dstb.at[j]], add=True)

        @pl.when(wid < XTRA)
        def _():
            pltpu.sync_copy(ones, acc.at[dstb.at[PCH]], add=True)

        plsc.subcore_barrier()
        pltpu.sync_copy(acc.at[pl.ds(s * RPT, RPT)], zbuf)
        pltpu.sync_copy(zbuf, out_hbm.at[c, pl.ds(s * RPT, RPT)])

    return _deg_sc


@functools.cache
def _make_msg_sc(F):
    @functools.partial(
        pl.kernel,
        mesh=_mesh(),
        out_type=jax.ShapeDtypeStruct((2, NP, F), _F32),
        compiler_params=_SC_PARAMS,
        scratch_types=(
            [pltpu.VMEM((PCH + 1, CH), jnp.int32),
             pltpu.VMEM((PCH + 1, CH), jnp.int32)]
            + [pltpu.VMEM((CH, F), _F32) for _ in range(NBUF)]
            + [pltpu.SemaphoreType.DMA for _ in range(2 * NBUF)]
            + [pltpu.VMEM_SHARED((NP, F), _F32)]
        ),
    )
    def _msg(edge_hbm, tab_hbm, out_hbm, srcb, dstb, *rest):
        gbs = rest[:NBUF]
        gsems = rest[NBUF:2 * NBUF]
        ssems = rest[2 * NBUF:3 * NBUF]
        acc = rest[3 * NBUF]
        c = lax.axis_index("c")
        s = lax.axis_index("s")
        wid = s * 2 + c
        cp_s = pltpu.async_copy(edge_hbm.at[0, pl.ds(wid * PCH, PCH), :],
                                srcb.at[pl.ds(0, PCH)], gsems[0])
        cp_d = pltpu.async_copy(edge_hbm.at[1, pl.ds(wid * PCH, PCH), :],
                                dstb.at[pl.ds(0, PCH)], gsems[1])
        z16 = jnp.zeros((16,), _F32)

        @pl.loop(0, CH)
        def _(i):
            for jj in range(0, F, 16):
                gbs[0][i, pl.ds(jj, 16)] = z16

        @pl.loop(0, RPT, step=CH)
        def _(r):
            pltpu.sync_copy(gbs[0], acc.at[pl.ds(s * RPT + r, CH), :])

        cp_s.wait()
        cp_d.wait()

        @pl.when(wid < XTRA)
        def _():
            pltpu.sync_copy(edge_hbm.at[0, pl.ds(32 * PCH + wid, 1), :],
                            srcb.at[pl.ds(PCH, 1)])
            pltpu.sync_copy(edge_hbm.at[1, pl.ds(32 * PCH + wid, 1), :],
                            dstb.at[pl.ds(PCH, 1)])

        plsc.subcore_barrier()

        @pl.loop(0, PCH, step=NBUF)
        def _(j):
            hs = [pltpu.async_copy(tab_hbm.at[srcb.at[j + k]], gbs[k],
                                   gsems[k]) for k in range(NBUF)]
            ss = []
            for k in range(NBUF):
                hs[k].wait()
                ss.append(pltpu.async_copy(gbs[k], acc.at[dstb.at[j + k]],
                                           ssems[k], add=True))
            for k in range(NBUF):
                ss[k].wait()

        @pl.when(wid < XTRA)
        def _():
            pltpu.sync_copy(tab_hbm.at[srcb.at[PCH]], gbs[0])
            pltpu.sync_copy(gbs[0], acc.at[dstb.at[PCH]], add=True)

        plsc.subcore_barrier()

        @pl.loop(0, RPT, step=CH)
        def _(r):
            pltpu.sync_copy(acc.at[pl.ds(s * RPT + r, CH), :], gbs[0])
            pltpu.sync_copy(gbs[0], out_hbm.at[c, pl.ds(s * RPT + r, CH), :])

    return _msg


def _dinv_col(d_ref):
    deg = d_ref[0:1, :] + d_ref[1:2, :] + 1.0
    dinv = lax.rsqrt(jnp.maximum(deg, 1e-12))
    return jnp.transpose(dinv, (1, 0))


def _tc1_body(x_ref, w_ref, d_ref, out_ref):
    dinv = _dinv_col(d_ref)
    h = _dot_bf16(x_ref[...], w_ref[...])
    out_ref[...] = h * dinv


def _tc1(x_pad, W1, degp):
    return pl.pallas_call(
        _tc1_body,
        grid=(NBLK,),
        in_specs=[
            pl.BlockSpec((BLK, D), lambda i: (i, 0)),
            pl.BlockSpec((D, H1), lambda i: (0, 0)),
            pl.BlockSpec((2, BLK), lambda i: (0, i)),
        ],
        out_specs=pl.BlockSpec((BLK, H1), lambda i: (i, 0)),
        out_shape=jax.ShapeDtypeStruct((NP, H1), _F32),
    )(x_pad, W1, degp)


def _norm_relu(a0_ref, a1_ref, hp_ref, d_ref, g_ref, be_ref, b_ref):
    dinv = _dinv_col(d_ref)
    sfeat = (a0_ref[0] + a1_ref[0] + hp_ref[...]) * dinv + b_ref[...]
    mu = jnp.mean(sfeat, axis=1, keepdims=True)
    var = jnp.mean((sfeat - mu) ** 2, axis=1, keepdims=True)
    y = (sfeat - mu) * lax.rsqrt(var + 1e-5) * g_ref[...] + be_ref[...]
    return jnp.maximum(y, 0.0), dinv


def _tc2_body(a0_ref, a1_ref, hp_ref, d_ref, g_ref, be_ref, b_ref,
              out_ref):
    y, dinv = _norm_relu(a0_ref, a1_ref, hp_ref, d_ref, g_ref, be_ref, b_ref)
    y16 = y.astype(_BF16).astype(_F32)
    out_ref[...] = y16 * dinv


def _tc2(accp1, hp1, degp, g1, be1, b1):
    return pl.pallas_call(
        _tc2_body,
        grid=(NBLK,),
        in_specs=[
            pl.BlockSpec((1, BLK, H1), lambda i: (0, i, 0)),
            pl.BlockSpec((1, BLK, H1), lambda i: (1, i, 0)),
            pl.BlockSpec((BLK, H1), lambda i: (i, 0)),
            pl.BlockSpec((2, BLK), lambda i: (0, i)),
            pl.BlockSpec((1, H1), lambda i: (0, 0)),
            pl.BlockSpec((1, H1), lambda i: (0, 0)),
            pl.BlockSpec((1, H1), lambda i: (0, 0)),
        ],
        out_specs=pl.BlockSpec((BLK, H1), lambda i: (i, 0)),
        out_shape=jax.ShapeDtypeStruct((NP, H1), _F32),
    )(accp1, accp1, hp1, degp, g1, be1, b1)


def _tc3_body(a0_ref, a1_ref, hp_ref, d_ref, w2_ref, g_ref, be_ref,
              b_ref, bt_ref, wc1_ref, bc1_ref, wc2t_ref, bc2_ref, out_ref,
              sums_ref, cnts_ref):
    i = pl.program_id(0)
    dinv = _dinv_col(d_ref)
    agg = (a0_ref[0] + a1_ref[0] + hp_ref[...]) * dinv
    w2b = w2_ref[...].astype(_BF16).astype(_F32)
    sfeat = jnp.dot(agg, w2b, preferred_element_type=_F32,
                    precision=_HIGH) + b_ref[...]
    mu = jnp.mean(sfeat, axis=1, keepdims=True)
    var = jnp.mean((sfeat - mu) ** 2, axis=1, keepdims=True)
    y = (sfeat - mu) * lax.rsqrt(var + 1e-5) * g_ref[...] + be_ref[...]
    y = jnp.maximum(y, 0.0)
    bb = bt_ref[0]
    gid = lax.broadcasted_iota(jnp.int32, (G, BLK), 0)
    oh = (gid == bb).astype(_F32)
    psum = jnp.dot(oh, y, preferred_element_type=_F32, precision=_HIGH)
    pcnt = jnp.sum(oh, axis=1, keepdims=True)

    @pl.when(i == 0)
    def _():
        sums_ref[...] = psum
        cnts_ref[...] = pcnt

    @pl.when(i > 0)
    def _():
        sums_ref[...] += psum
        cnts_ref[...] += pcnt

    @pl.when(i == NBLK - 1)
    def _():
        pooled = sums_ref[...] / jnp.maximum(cnts_ref[...], 1.0)
        z = _dot_bf16(pooled, wc1_ref[...]) + bc1_ref[...]
        z = jnp.maximum(z, 0.0)
        zb = z.astype(_BF16).astype(_F32)
        wb = wc2t_ref[...].astype(_BF16).astype(_F32)
        out_ref[...] = (jnp.sum(zb * wb, axis=1, keepdims=True)
                        + bc2_ref[...])


def _tc3(accp2, hp2, degp, W2, g2, be2, b2, batch3, Wc1, bc1, Wc2t, bc2):
    return pl.pallas_call(
        _tc3_body,
        grid=(NBLK,),
        in_specs=[
            pl.BlockSpec((1, BLK, H1), lambda i: (0, i, 0)),
            pl.BlockSpec((1, BLK, H1), lambda i: (1, i, 0)),
            pl.BlockSpec((BLK, H1), lambda i: (i, 0)),
            pl.BlockSpec((2, BLK), lambda i: (0, i)),
            pl.BlockSpec((H1, H2), lambda i: (0, 0)),
            pl.BlockSpec((1, H2), lambda i: (0, 0)),
            pl.BlockSpec((1, H2), lambda i: (0, 0)),
            pl.BlockSpec((1, H2), lambda i: (0, 0)),
            pl.BlockSpec((1, 1, BLK), lambda i: (i, 0, 0)),
            pl.BlockSpec((H2, 32), lambda i: (0, 0)),
            pl.BlockSpec((1, 32), lambda i: (0, 0)),
            pl.BlockSpec((1, 32), lambda i: (0, 0)),
            pl.BlockSpec((1, 1), lambda i: (0, 0)),
        ],
        out_specs=pl.BlockSpec((G, 1), lambda i: (0, 0)),
        out_shape=jax.ShapeDtypeStruct((G, 1), _F32),
        scratch_shapes=[
            pltpu.VMEM((G, H2), _F32),
            pltpu.VMEM((G, 1), _F32),
        ],
    )(accp2, accp2, hp2, degp, W2, g2, be2, b2, batch3, Wc1, bc1,
      Wc2t, bc2)


def kernel(x, edge_index, batch, W1, b1, g1, be1, W2, b2, g2, be2,
           Wc1, bc1, Wc2, bc2):
    edge3 = edge_index.reshape(2, NCHT, CH)
    x_pad = jnp.pad(x, ((0, NP - N), (0, 0)))
    batch3 = jnp.pad(batch, (0, NP - N),
                     constant_values=G).reshape(NBLK, 1, BLK)

    degp = _make_deg_sc()(edge3)
    hp1 = _tc1(x_pad, W1, degp)
    accp1 = _make_msg_sc(H1)(edge3, hp1)
    hp2 = _tc2(accp1, hp1, degp, g1.reshape(1, H1), be1.reshape(1, H1),
               b1.reshape(1, H1))
    accp2 = _make_msg_sc(H1)(edge3, hp2)
    return _tc3(accp2, hp2, degp, W2, g2.reshape(1, H2), be2.reshape(1, H2),
                b2.reshape(1, H2), batch3, Wc1, bc1.reshape(1, 32),
                Wc2.reshape(1, 32), bc2.reshape(1, 1))
